# baseline (speedup 1.0000x reference)
import functools

import jax
import jax.numpy as jnp
from jax import lax
from jax.experimental import pallas as pl
from jax.experimental.pallas import tpu as pltpu
from jax.experimental.pallas import tpu_sc as plsc

BSZ = 16
SEQ = 2048
HID = 128
NSEG = 512
CHUNK = 128
NCHUNK = SEQ // CHUNK
HGRP = HID // 16


def _tagop_body(values_hbm, scores_hbm, index_hbm, mean_out, max_out,
                idx_v, sc_v, vals_v, acc_v, cnt_v, mx_v, mx2_v, mxf_v):
    c = lax.axis_index("c")
    s = lax.axis_index("s")
    wid = s * 2 + c

    @pl.when(wid < BSZ)
    def _run():
        b = wid
        pltpu.sync_copy(index_hbm.at[b], idx_v)
        pltpu.sync_copy(scores_hbm.at[b], sc_v)

        zero = jnp.zeros((16,), jnp.float32)
        neg = jnp.full((16,), -jnp.inf, jnp.float32)

        def zbody(r, _):
            for h in range(HGRP):
                acc_v[r, pl.ds(h * 16, 16)] = zero
            cnt_v[pl.ds(r * 16, 16)] = zero
            mx_v[pl.ds(r * 16, 16)] = neg
            mx2_v[pl.ds(r * 16, 16)] = neg
            return 0

        lax.fori_loop(0, NSEG, zbody, 0)

        def chunk_body(ck, _):
            pltpu.sync_copy(values_hbm.at[b, pl.ds(ck * CHUNK, CHUNK)], vals_v)
            base = ck * CHUNK
            one = jnp.ones((16,), jnp.float32)

            @plsc.parallel_loop(0, CHUNK // 16)
            def add_pass(g):
                gb = base + g * 16
                iv = idx_v[pl.ds(gb, 16)]
                tb = g * 16
                for j in range(16):
                    i = iv[j]
                    for h in range(HGRP):
                        col = pl.ds(h * 16, 16)
                        plsc.addupdate(acc_v.at[i, col], vals_v[tb + j, col])
                    plsc.addupdate(cnt_v.at[pl.ds(i * 16, 16)], one)

            def mx_body(g, _):
                gb = base + g * 16
                iv = idx_v[pl.ds(gb, 16)]
                sv16 = sc_v[pl.ds(gb, 16)]
                for j in range(16):
                    i = iv[j]
                    ci = pl.ds(i * 16, 16)
                    sv = jnp.full((16,), sv16[j], jnp.float32)
                    mref = mx_v if j % 2 == 0 else mx2_v
                    mref[ci] = jnp.maximum(mref[ci], sv)
                return 0

            lax.fori_loop(0, CHUNK // 16, mx_body, 0)
            return 0

        lax.fori_loop(0, NCHUNK, chunk_body, 0)

        def fbody(r, _):
            recip = 1.0 / jnp.maximum(cnt_v[pl.ds(r * 16, 16)], 1.0)
            for h in range(HGRP):
                col = pl.ds(h * 16, 16)
                acc_v[r, col] = acc_v[r, col] * recip
            return 0

        lax.fori_loop(0, NSEG, fbody, 0)

        lanes = lax.iota(jnp.int32, 16)

        def gbody(g, _):
            m = jnp.zeros((16,), jnp.float32)
            cz = jnp.zeros((16,), jnp.float32)
            for j in range(16):
                r = g * 16 + j
                sel = lanes == j
                rr = pl.ds(r * 16, 16)
                m = jnp.where(sel, jnp.maximum(mx_v[rr], mx2_v[rr]), m)
                cz = jnp.where(sel, cnt_v[rr], cz)
            mxf_v[pl.ds(g * 16, 16)] = jnp.where(cz > 0.0, m, 0.0)
            return 0

        lax.fori_loop(0, NSEG // 16, gbody, 0)

        pltpu.sync_copy(acc_v, mean_out.at[b])
        pltpu.sync_copy(mxf_v, max_out.at[b])


@jax.jit
def _tagop(values, scores, index):
    mesh = plsc.VectorSubcoreMesh(core_axis_name="c", subcore_axis_name="s")
    fn = functools.partial(
        pl.kernel,
        mesh=mesh,
        out_type=(
            jax.ShapeDtypeStruct((BSZ, NSEG, HID), jnp.float32),
            jax.ShapeDtypeStruct((BSZ, NSEG), jnp.float32),
        ),
        scratch_types=[
            pltpu.VMEM((SEQ,), jnp.int32),
            pltpu.VMEM((SEQ,), jnp.float32),
            pltpu.VMEM((CHUNK, HID), jnp.float32),
            pltpu.VMEM((NSEG, HID), jnp.float32),
            pltpu.VMEM((NSEG * 16,), jnp.float32),
            pltpu.VMEM((NSEG * 16,), jnp.float32),
            pltpu.VMEM((NSEG * 16,), jnp.float32),
            pltpu.VMEM((NSEG,), jnp.float32),
        ],
    )(_tagop_body)
    return fn(values, scores, index)


def kernel(values, scores, index):
    return _tagop(values, scores, index)


# speedup vs baseline: 1.1239x; 1.1239x over previous
import functools

import jax
import jax.numpy as jnp
from jax import lax
from jax.experimental import pallas as pl
from jax.experimental.pallas import tpu as pltpu
from jax.experimental.pallas import tpu_sc as plsc

BSZ = 16
SEQ = 2048
HID = 128
NSEG = 512
CHUNK = 128
NCHUNK = SEQ // CHUNK
GA = 5
GB = 3
COFF_B = GA * 16


def _run_role(b, ng, coff, do_mx, values_hbm, scores_hbm, index_hbm,
              mean_out, max_out, idx_v, sc_v, vals_v, acc_v, cnt_v,
              mx_v, mx2_v, mxf_v):
    ncol = ng * 16
    pltpu.sync_copy(index_hbm.at[b], idx_v)
    if do_mx:
        pltpu.sync_copy(scores_hbm.at[b], sc_v)

    zero = jnp.zeros((16,), jnp.float32)
    neg = jnp.full((16,), -jnp.inf, jnp.float32)

    def zbody(r, _):
        for h in range(ng):
            acc_v[pl.ds(r * ncol + h * 16, 16)] = zero
        cnt_v[pl.ds(r * 16, 16)] = zero
        if do_mx:
            mx_v[pl.ds(r * 16, 16)] = neg
            mx2_v[pl.ds(r * 16, 16)] = neg
        return 0

    lax.fori_loop(0, NSEG, zbody, 0)

    def chunk_body(ck, _):
        pltpu.sync_copy(values_hbm.at[b, pl.ds(ck * CHUNK, CHUNK)], vals_v)
        base = ck * CHUNK
        one = jnp.ones((16,), jnp.float32)

        def grp_body(g, _):
            gb = base + g * 16
            iv = idx_v[pl.ds(gb, 16)]
            if do_mx:
                sv16 = sc_v[pl.ds(gb, 16)]
            tb = g * 16

            def ldrow(j):
                return [vals_v[tb + j, pl.ds(coff + h * 16, 16)]
                        for h in range(ng)]

            vrow = ldrow(0)
            for j in range(16):
                nxt = ldrow(j + 1) if j < 15 else None
                i = iv[j]
                rb = i * ncol
                for h in range(ng):
                    plsc.addupdate(acc_v.at[pl.ds(rb + h * 16, 16)], vrow[h])
                ci = pl.ds(i * 16, 16)
                plsc.addupdate(cnt_v.at[ci], one)
                if do_mx:
                    sv = jnp.full((16,), sv16[j], jnp.float32)
                    mref = mx_v if j % 2 == 0 else mx2_v
                    mref[ci] = jnp.maximum(mref[ci], sv)
                vrow = nxt
            return 0

        lax.fori_loop(0, CHUNK // 16, grp_body, 0)
        return 0

    lax.fori_loop(0, NCHUNK, chunk_body, 0)

    def fbody(r, _):
        recip = 1.0 / jnp.maximum(cnt_v[pl.ds(r * 16, 16)], 1.0)
        for h in range(ng):
            col = pl.ds(r * ncol + h * 16, 16)
            acc_v[col] = acc_v[col] * recip
        return 0

    lax.fori_loop(0, NSEG, fbody, 0)

    if do_mx:
        lanes = lax.iota(jnp.int32, 16)

        def gbody(g, _):
            m = jnp.zeros((16,), jnp.float32)
            cz = jnp.zeros((16,), jnp.float32)
            for j in range(16):
                r = g * 16 + j
                sel = lanes == j
                rr = pl.ds(r * 16, 16)
                m = jnp.where(sel, jnp.maximum(mx_v[rr], mx2_v[rr]), m)
                cz = jnp.where(sel, cnt_v[rr], cz)
            mxf_v[pl.ds(g * 16, 16)] = jnp.where(cz > 0.0, m, 0.0)
            return 0

        lax.fori_loop(0, NSEG // 16, gbody, 0)
        pltpu.sync_copy(mxf_v, max_out.at[b])

    pltpu.sync_copy(acc_v, mean_out.at[b])


def _tagop_body(values_hbm, scores_hbm, index_hbm, mean_a_out, mean_b_out,
                max_out, idx_v, sc_v, vals_v, acc_a_v, acc_b_v, cnt_v,
                mx_v, mx2_v, mxf_v):
    c = lax.axis_index("c")
    s = lax.axis_index("s")
    b = s

    @pl.when(c == 0)
    def _a():
        _run_role(b, GA, 0, False, values_hbm, scores_hbm, index_hbm,
                  mean_a_out, max_out, idx_v, sc_v, vals_v, acc_a_v, cnt_v,
                  mx_v, mx2_v, mxf_v)

    @pl.when(c == 1)
    def _b():
        _run_role(b, GB, COFF_B, True, values_hbm, scores_hbm, index_hbm,
                  mean_b_out, max_out, idx_v, sc_v, vals_v, acc_b_v, cnt_v,
                  mx_v, mx2_v, mxf_v)


@jax.jit
def _tagop(values, scores, index):
    mesh = plsc.VectorSubcoreMesh(core_axis_name="c", subcore_axis_name="s")
    fn = functools.partial(
        pl.kernel,
        mesh=mesh,
        out_type=(
            jax.ShapeDtypeStruct((BSZ, NSEG * GA * 16), jnp.float32),
            jax.ShapeDtypeStruct((BSZ, NSEG * GB * 16), jnp.float32),
            jax.ShapeDtypeStruct((BSZ, NSEG), jnp.float32),
        ),
        scratch_types=[
            pltpu.VMEM((SEQ,), jnp.int32),
            pltpu.VMEM((SEQ,), jnp.float32),
            pltpu.VMEM((CHUNK, HID), jnp.float32),
            pltpu.VMEM((NSEG * GA * 16,), jnp.float32),
            pltpu.VMEM((NSEG * GB * 16,), jnp.float32),
            pltpu.VMEM((NSEG * 16,), jnp.float32),
            pltpu.VMEM((NSEG * 16,), jnp.float32),
            pltpu.VMEM((NSEG * 16,), jnp.float32),
            pltpu.VMEM((NSEG,), jnp.float32),
        ],
    )(_tagop_body)
    mean_a, mean_b, max_scores = fn(values, scores, index)
    mean_vec = jnp.concatenate(
        [mean_a.reshape(BSZ, NSEG, GA * 16), mean_b.reshape(BSZ, NSEG, GB * 16)],
        axis=2)
    return mean_vec, max_scores


def kernel(values, scores, index):
    return _tagop(values, scores, index)


# speedup vs baseline: 1.1894x; 1.0582x over previous
import functools

import jax
import jax.numpy as jnp
from jax import lax
from jax.experimental import pallas as pl
from jax.experimental.pallas import tpu as pltpu
from jax.experimental.pallas import tpu_sc as plsc

BSZ = 16
SEQ = 2048
HID = 128
NSEG = 512
CHUNK = 64
NCHUNK = SEQ // CHUNK
GA = 5
GB = 3
COFF_B = GA * 16


def _run_role(b, ng, coff, do_mx, values_hbm, scores_hbm, index_hbm,
              mean_out, max_out, idx_v, sc_v, vals0_v, vals1_v, acc_v,
              cnt_v, mx_v, mx2_v, mxf_v, sem0, sem1):
    ncol = ng * 16
    pltpu.sync_copy(index_hbm.at[b], idx_v)
    if do_mx:
        pltpu.sync_copy(scores_hbm.at[b], sc_v)

    zero = jnp.zeros((16,), jnp.float32)
    neg = jnp.full((16,), -jnp.inf, jnp.float32)

    def zbody(r, _):
        for h in range(ng):
            acc_v[pl.ds(r * ncol + h * 16, 16)] = zero
        cnt_v[pl.ds(r * 16, 16)] = zero
        if do_mx:
            mx_v[pl.ds(r * 16, 16)] = neg
            mx2_v[pl.ds(r * 16, 16)] = neg
        return 0

    def start(ck, buf, sem):
        pltpu.make_async_copy(
            values_hbm.at[b, pl.ds(ck * CHUNK, CHUNK)], buf, sem).start()

    def wait(buf, sem):
        pltpu.make_async_copy(
            values_hbm.at[b, pl.ds(0, CHUNK)], buf, sem).wait()

    start(0, vals0_v, sem0)
    lax.fori_loop(0, NSEG, zbody, 0)

    def compute_chunk(ck, vals_v):
        base = ck * CHUNK
        one = jnp.ones((16,), jnp.float32)

        def grp_body(g, _):
            gb = base + g * 16
            iv = idx_v[pl.ds(gb, 16)]
            if do_mx:
                sv16 = sc_v[pl.ds(gb, 16)]
            tb = g * 16

            def ldrow(j):
                return [vals_v[tb + j, pl.ds(coff + h * 16, 16)]
                        for h in range(ng)]

            vrow = ldrow(0)
            for j in range(16):
                nxt = ldrow(j + 1) if j < 15 else None
                i = iv[j]
                rb = i * ncol
                for h in range(ng):
                    plsc.addupdate(acc_v.at[pl.ds(rb + h * 16, 16)], vrow[h])
                ci = pl.ds(i * 16, 16)
                plsc.addupdate(cnt_v.at[ci], one)
                if do_mx:
                    sv = jnp.full((16,), sv16[j], jnp.float32)
                    mref = mx_v if j % 2 == 0 else mx2_v
                    mref[ci] = jnp.maximum(mref[ci], sv)
                vrow = nxt
            return 0

        lax.fori_loop(0, CHUNK // 16, grp_body, 0)

    def chunk_pair(k2, _):
        ck = k2 * 2
        wait(vals0_v, sem0)
        start(ck + 1, vals1_v, sem1)
        compute_chunk(ck, vals0_v)
        wait(vals1_v, sem1)

        @pl.when(ck + 2 < NCHUNK)
        def _():
            start(ck + 2, vals0_v, sem0)

        compute_chunk(ck + 1, vals1_v)
        return 0

    lax.fori_loop(0, NCHUNK // 2, chunk_pair, 0)

    def fbody(r, _):
        recip = 1.0 / jnp.maximum(cnt_v[pl.ds(r * 16, 16)], 1.0)
        for h in range(ng):
            col = pl.ds(r * ncol + h * 16, 16)
            acc_v[col] = acc_v[col] * recip
        return 0

    lax.fori_loop(0, NSEG, fbody, 0)

    if do_mx:
        lanes = lax.iota(jnp.int32, 16)

        def gbody(g, _):
            m = jnp.zeros((16,), jnp.float32)
            cz = jnp.zeros((16,), jnp.float32)
            for j in range(16):
                r = g * 16 + j
                sel = lanes == j
                rr = pl.ds(r * 16, 16)
                m = jnp.where(sel, jnp.maximum(mx_v[rr], mx2_v[rr]), m)
                cz = jnp.where(sel, cnt_v[rr], cz)
            mxf_v[pl.ds(g * 16, 16)] = jnp.where(cz > 0.0, m, 0.0)
            return 0

        lax.fori_loop(0, NSEG // 16, gbody, 0)
        pltpu.sync_copy(mxf_v, max_out.at[b])

    pltpu.sync_copy(acc_v, mean_out.at[b])


def _tagop_body(values_hbm, scores_hbm, index_hbm, mean_a_out, mean_b_out,
                max_out, idx_v, sc_v, vals0_v, vals1_v, acc_a_v, acc_b_v,
                cnt_v, mx_v, mx2_v, mxf_v, sem0, sem1):
    c = lax.axis_index("c")
    s = lax.axis_index("s")
    b = s

    @pl.when(c == 0)
    def _a():
        _run_role(b, GA, 0, False, values_hbm, scores_hbm, index_hbm,
                  mean_a_out, max_out, idx_v, sc_v, vals0_v, vals1_v,
                  acc_a_v, cnt_v, mx_v, mx2_v, mxf_v, sem0, sem1)

    @pl.when(c == 1)
    def _b():
        _run_role(b, GB, COFF_B, True, values_hbm, scores_hbm, index_hbm,
                  mean_b_out, max_out, idx_v, sc_v, vals0_v, vals1_v,
                  acc_b_v, cnt_v, mx_v, mx2_v, mxf_v, sem0, sem1)


@jax.jit
def _tagop(values, scores, index):
    mesh = plsc.VectorSubcoreMesh(core_axis_name="c", subcore_axis_name="s")
    fn = functools.partial(
        pl.kernel,
        mesh=mesh,
        out_type=(
            jax.ShapeDtypeStruct((BSZ, NSEG * GA * 16), jnp.float32),
            jax.ShapeDtypeStruct((BSZ, NSEG * GB * 16), jnp.float32),
            jax.ShapeDtypeStruct((BSZ, NSEG), jnp.float32),
        ),
        scratch_types=[
            pltpu.VMEM((SEQ,), jnp.int32),
            pltpu.VMEM((SEQ,), jnp.float32),
            pltpu.VMEM((CHUNK, HID), jnp.float32),
            pltpu.VMEM((CHUNK, HID), jnp.float32),
            pltpu.VMEM((NSEG * GA * 16,), jnp.float32),
            pltpu.VMEM((NSEG * GB * 16,), jnp.float32),
            pltpu.VMEM((NSEG * 16,), jnp.float32),
            pltpu.VMEM((NSEG * 16,), jnp.float32),
            pltpu.VMEM((NSEG * 16,), jnp.float32),
            pltpu.VMEM((NSEG,), jnp.float32),
            pltpu.SemaphoreType.DMA,
            pltpu.SemaphoreType.DMA,
        ],
    )(_tagop_body)
    mean_a, mean_b, max_scores = fn(values, scores, index)
    mean_vec = jnp.concatenate(
        [mean_a.reshape(BSZ, NSEG, GA * 16), mean_b.reshape(BSZ, NSEG, GB * 16)],
        axis=2)
    return mean_vec, max_scores


def kernel(values, scores, index):
    return _tagop(values, scores, index)


# speedup vs baseline: 1.7486x; 1.4702x over previous
import functools

import jax
import jax.numpy as jnp
from jax import lax
from jax.experimental import pallas as pl
from jax.experimental.pallas import tpu as pltpu
from jax.experimental.pallas import tpu_sc as plsc

BSZ = 16
SEQ = 2048
HID = 128
NSEG = 512
CHUNK = 128
HALF = SEQ // 2
NCH = HALF // CHUNK


def _tagop_body(values_hbm, scores_hbm, index2_hbm, mean_out, max_out,
                idx2_v, sc_v, vals0_v, vals1_v, mx_v, mx2_v, cnt_v, cntp_v,
                ca_v, cb_v, mxf_v, acc_sh, mx_sh, cnt_sh, sem0, sem1, sems):
    c = lax.axis_index("c")
    s = lax.axis_index("s")
    bl = s // 2
    h = s % 2
    b = c * 8 + bl
    slot = s
    boff = bl * NSEG

    pltpu.sync_copy(index2_hbm.at[b, pl.ds(h * NCH, NCH)], idx2_v)
    pltpu.sync_copy(scores_hbm.at[b, pl.ds(h * HALF, HALF)], sc_v)

    zero = jnp.zeros((16,), jnp.float32)
    neg = jnp.full((16,), -jnp.inf, jnp.float32)

    def zv(r, _):
        for k in range(8):
            vals0_v[r, pl.ds(k * 16, 16)] = zero
        return 0

    lax.fori_loop(0, CHUNK, zv, 0)
    zbase = boff + h * 256
    pltpu.sync_copy(vals0_v, acc_sh.at[pl.ds(zbase, 128)])
    pltpu.sync_copy(vals0_v, acc_sh.at[pl.ds(zbase + 128, 128)])

    bv = jnp.full((16,), boff, jnp.int32)

    def ob(r, _):
        for k in range(8):
            col = pl.ds(k * 16, 16)
            idx2_v[r, col] = idx2_v[r, col] + bv
        return 0

    lax.fori_loop(0, NCH, ob, 0)

    def zm(r, _):
        rr = pl.ds(r * 16, 16)
        mx_v[rr] = neg
        mx2_v[rr] = neg
        cnt_v[rr] = zero
        return 0

    lax.fori_loop(0, NSEG, zm, 0)

    plsc.subcore_barrier()

    def start_load(ck, buf, sem):
        pltpu.make_async_copy(
            values_hbm.at[b, pl.ds(h * HALF + ck * CHUNK, CHUNK)],
            buf, sem).start()

    def wait_load(buf, sem):
        pltpu.make_async_copy(
            values_hbm.at[b, pl.ds(0, CHUNK)], buf, sem).wait()

    start_load(0, vals0_v, sem0)
    one = jnp.ones((16,), jnp.float32)

    def mx_chunk(ck):
        def grp(g, _):
            r = ck * (CHUNK // 16) + g
            iv = idx2_v[r // 8, pl.ds((r % 8) * 16, 16)]
            sv16 = sc_v[pl.ds(r * 16, 16)]
            for j in range(16):
                i = iv[j] - boff
                ci = pl.ds(i * 16, 16)
                plsc.addupdate(cnt_v.at[ci], one)
                sv = jnp.full((16,), sv16[j], jnp.float32)
                mref = mx_v if j % 2 == 0 else mx2_v
                mref[ci] = jnp.maximum(mref[ci], sv)
            return 0

        lax.fori_loop(0, CHUNK // 16, grp, 0)

    def chunk_pair(k2, _):
        ck = k2 * 2
        wait_load(vals0_v, sem0)
        start_load(ck + 1, vals1_v, sem1)
        cp0 = pltpu.async_copy(
            vals0_v, acc_sh.at[idx2_v.at[ck]], sems, add=True)
        mx_chunk(ck)
        cp0.wait()

        wait_load(vals1_v, sem1)

        @pl.when(ck + 2 < NCH)
        def _():
            start_load(ck + 2, vals0_v, sem0)

        cp1 = pltpu.async_copy(
            vals1_v, acc_sh.at[idx2_v.at[ck + 1]], sems, add=True)
        mx_chunk(ck + 1)
        cp1.wait()
        return 0

    lax.fori_loop(0, NCH // 2, chunk_pair, 0)

    def mm(r, _):
        rr = pl.ds(r * 16, 16)
        mx_v[rr] = jnp.maximum(mx_v[rr], mx2_v[rr])
        return 0

    lax.fori_loop(0, NSEG, mm, 0)
    pltpu.sync_copy(mx_v, mx_sh.at[slot])
    pltpu.sync_copy(cnt_v, cnt_sh.at[slot])

    plsc.subcore_barrier()

    pslot = slot - 2 * h + 1
    for p in range(2):
        seg0 = h * 256 + p * 128
        fin = vals0_v if p == 0 else vals1_v
        pltpu.sync_copy(acc_sh.at[pl.ds(boff + seg0, 128)], fin)
        pltpu.sync_copy(cnt_sh.at[slot, pl.ds(seg0 * 16, 2048)], ca_v)
        pltpu.sync_copy(cnt_sh.at[pslot, pl.ds(seg0 * 16, 2048)], cb_v)

        def fb(r, _):
            rr = pl.ds(r * 16, 16)
            tot = ca_v[rr] + cb_v[rr]
            recip = 1.0 / jnp.maximum(tot, 1.0)
            for k in range(8):
                col = pl.ds(k * 16, 16)
                fin[r, col] = fin[r, col] * recip
            return 0

        lax.fori_loop(0, CHUNK, fb, 0)
        pltpu.sync_copy(fin, mean_out.at[b, pl.ds(seg0, 128)])

    @pl.when(h == 0)
    def _fmax():
        pltpu.sync_copy(mx_sh.at[pslot], mx2_v)
        pltpu.sync_copy(cnt_sh.at[pslot], cntp_v)
        lanes = lax.iota(jnp.int32, 16)

        def gb(g, _):
            m = jnp.zeros((16,), jnp.float32)
            cz = jnp.zeros((16,), jnp.float32)
            for j in range(16):
                rr = pl.ds((g * 16 + j) * 16, 16)
                sel = lanes == j
                m = jnp.where(sel, jnp.maximum(mx_v[rr], mx2_v[rr]), m)
                cz = jnp.where(sel, cnt_v[rr] + cntp_v[rr], cz)
            mxf_v[pl.ds(g * 16, 16)] = jnp.where(cz > 0.0, m, 0.0)
            return 0

        lax.fori_loop(0, NSEG // 16, gb, 0)
        pltpu.sync_copy(mxf_v, max_out.at[b])


@jax.jit
def _tagop(values, scores, index):
    mesh = plsc.VectorSubcoreMesh(core_axis_name="c", subcore_axis_name="s")
    fn = functools.partial(
        pl.kernel,
        mesh=mesh,
        out_type=(
            jax.ShapeDtypeStruct((BSZ, NSEG, HID), jnp.float32),
            jax.ShapeDtypeStruct((BSZ, NSEG), jnp.float32),
        ),
        scratch_types=[
            pltpu.VMEM((NCH, CHUNK), jnp.int32),
            pltpu.VMEM((HALF,), jnp.float32),
            pltpu.VMEM((CHUNK, HID), jnp.float32),
            pltpu.VMEM((CHUNK, HID), jnp.float32),
            pltpu.VMEM((NSEG * 16,), jnp.float32),
            pltpu.VMEM((NSEG * 16,), jnp.float32),
            pltpu.VMEM((NSEG * 16,), jnp.float32),
            pltpu.VMEM((NSEG * 16,), jnp.float32),
            pltpu.VMEM((2048,), jnp.float32),
            pltpu.VMEM((2048,), jnp.float32),
            pltpu.VMEM((NSEG,), jnp.float32),
            pltpu.VMEM_SHARED((8 * NSEG, HID), jnp.float32),
            pltpu.VMEM_SHARED((16, NSEG * 16), jnp.float32),
            pltpu.VMEM_SHARED((16, NSEG * 16), jnp.float32),
            pltpu.SemaphoreType.DMA,
            pltpu.SemaphoreType.DMA,
            pltpu.SemaphoreType.DMA,
        ],
    )(_tagop_body)
    idx2 = index.reshape(BSZ, SEQ // CHUNK, CHUNK)
    return fn(values, scores, idx2)


def kernel(values, scores, index):
    return _tagop(values, scores, index)


# speedup vs baseline: 1.9114x; 1.0931x over previous
import functools

import jax
import jax.numpy as jnp
from jax import lax
from jax.experimental import pallas as pl
from jax.experimental.pallas import tpu as pltpu
from jax.experimental.pallas import tpu_sc as plsc

BSZ = 16
SEQ = 2048
HID = 128
NSEG = 512
CHUNK = 128
IROW = 128
HALF = SEQ // 2
NCH = HALF // CHUNK
NIR = HALF // IROW


def _tagop_body(values_hbm, scores_hbm, index2_hbm, mean_out, max_out,
                idx2_v, sc_v, vals0_v, vals1_v, mx_v, mx2_v, cnt_v, cntp_v,
                mxf_v, acc_sh, mx_sh, cnt_sh, sem0, sem1, sems, semm):
    c = lax.axis_index("c")
    s = lax.axis_index("s")
    bl = s // 2
    h = s % 2
    b = c * 8 + bl
    slot = s
    boff = bl * NSEG

    pltpu.sync_copy(index2_hbm.at[b, pl.ds(h * NIR, NIR)], idx2_v)
    pltpu.sync_copy(scores_hbm.at[b, pl.ds(h * HALF, HALF)], sc_v)

    pltpu.make_async_copy(
        values_hbm.at[b, pl.ds(h * HALF, CHUNK)], vals1_v, sem1).start()

    zero = jnp.zeros((16,), jnp.float32)
    neg = jnp.full((16,), -jnp.inf, jnp.float32)

    def zv(r, _):
        for k in range(8):
            vals0_v[r, pl.ds(k * 16, 16)] = zero
        return 0

    lax.fori_loop(0, CHUNK, zv, 0)
    zbase = boff + h * 256
    pltpu.sync_copy(vals0_v, acc_sh.at[pl.ds(zbase, 128)])
    pltpu.sync_copy(vals0_v, acc_sh.at[pl.ds(zbase + 128, 128)])

    bv = jnp.full((16,), boff, jnp.int32)

    def ob(r, _):
        for k in range(8):
            col = pl.ds(k * 16, 16)
            idx2_v[r, col] = idx2_v[r, col] + bv
        return 0

    lax.fori_loop(0, NIR, ob, 0)

    def zm(r, _):
        rr = pl.ds(r * 16, 16)
        mx_v[rr] = neg
        mx2_v[rr] = neg
        cnt_v[rr] = zero
        return 0

    lax.fori_loop(0, NSEG, zm, 0)

    def start_load(ck, buf, sem):
        pltpu.make_async_copy(
            values_hbm.at[b, pl.ds(h * HALF + ck * CHUNK, CHUNK)],
            buf, sem).start()

    def wait_load(buf, sem):
        pltpu.make_async_copy(
            values_hbm.at[b, pl.ds(0, CHUNK)], buf, sem).wait()

    plsc.subcore_barrier()
    one = jnp.ones((16,), jnp.float32)

    def mx_chunk(ck):
        def grp(g, _):
            r = ck * (CHUNK // 16) + g
            iv = idx2_v[r // 8, pl.ds((r % 8) * 16, 16)]
            sv16 = sc_v[pl.ds(r * 16, 16)]
            for j in range(16):
                i = iv[j] - boff
                ci = pl.ds(i * 16, 16)
                plsc.addupdate(cnt_v.at[ci], one)
                sv = jnp.full((16,), sv16[j], jnp.float32)
                mref = mx_v if j % 2 == 0 else mx2_v
                mref[ci] = jnp.maximum(mref[ci], sv)
            return 0

        lax.fori_loop(0, CHUNK // 16, grp, 0)

    def chunk_pair(k2, _):
        ck = k2 * 2
        wait_load(vals1_v, sem1)
        start_load(ck + 1, vals0_v, sem0)
        cp0 = pltpu.async_copy(
            vals1_v, acc_sh.at[idx2_v.at[ck]], sems, add=True)
        mx_chunk(ck)
        cp0.wait()

        wait_load(vals0_v, sem0)

        @pl.when(ck + 2 < NCH)
        def _():
            start_load(ck + 2, vals1_v, sem1)

        cp1 = pltpu.async_copy(
            vals0_v, acc_sh.at[idx2_v.at[ck + 1]], sems, add=True)
        mx_chunk(ck + 1)
        cp1.wait()
        return 0

    lax.fori_loop(0, NCH // 2, chunk_pair, 0)

    def mm(r, _):
        rr = pl.ds(r * 16, 16)
        mx_v[rr] = jnp.maximum(mx_v[rr], mx2_v[rr])
        return 0

    lax.fori_loop(0, NSEG, mm, 0)
    ph = 1 - h
    pltpu.sync_copy(mx_v.at[pl.ds(ph * 4096, 4096)], mx_sh.at[slot])
    pltpu.sync_copy(cnt_v.at[pl.ds(ph * 4096, 4096)], cnt_sh.at[slot])

    plsc.subcore_barrier()

    pslot = slot - 2 * h + 1
    seg0 = h * 256
    cp_a = pltpu.async_copy(acc_sh.at[pl.ds(boff + seg0, 128)],
                            vals0_v, sem0)
    cp_b = pltpu.async_copy(acc_sh.at[pl.ds(boff + seg0 + 128, 128)],
                            vals1_v, sem1)
    cp_c = pltpu.async_copy(cnt_sh.at[pslot], cntp_v, sems)
    cp_d = pltpu.async_copy(mx_sh.at[pslot], mx2_v.at[pl.ds(0, 4096)], semm)
    cp_c.wait()
    outs = []
    for p in range(2):
        sp = seg0 + p * 128
        fin = vals0_v if p == 0 else vals1_v
        (cp_a if p == 0 else cp_b).wait()

        def fb(r, _):
            tot = (cnt_v[pl.ds((sp + r) * 16, 16)]
                   + cntp_v[pl.ds((p * 128 + r) * 16, 16)])
            recip = 1.0 / jnp.maximum(tot, 1.0)
            for k in range(8):
                col = pl.ds(k * 16, 16)
                fin[r, col] = fin[r, col] * recip
            return 0

        lax.fori_loop(0, 128, fb, 0)
        outs.append(pltpu.async_copy(
            fin, mean_out.at[b, pl.ds(sp, 128)], sem0 if p == 0 else sem1))

    cp_d.wait()
    lanes = lax.iota(jnp.int32, 16)
    sbase = h * 256

    def gb(g, _):
        m = jnp.zeros((16,), jnp.float32)
        cz = jnp.zeros((16,), jnp.float32)
        for j in range(16):
            r = g * 16 + j
            ro = pl.ds((sbase + r) * 16, 16)
            rp = pl.ds(r * 16, 16)
            sel = lanes == j
            m = jnp.where(sel, jnp.maximum(mx_v[ro], mx2_v[rp]), m)
            cz = jnp.where(sel, cnt_v[ro] + cntp_v[rp], cz)
        mxf_v[pl.ds(g * 16, 16)] = jnp.where(cz > 0.0, m, 0.0)
        return 0

    lax.fori_loop(0, 256 // 16, gb, 0)
    pltpu.sync_copy(mxf_v.at[pl.ds(0, 256)], max_out.at[b, pl.ds(sbase, 256)])
    outs[0].wait()
    outs[1].wait()


@jax.jit
def _tagop(values, scores, index):
    mesh = plsc.VectorSubcoreMesh(core_axis_name="c", subcore_axis_name="s")
    fn = functools.partial(
        pl.kernel,
        mesh=mesh,
        out_type=(
            jax.ShapeDtypeStruct((BSZ, NSEG, HID), jnp.float32),
            jax.ShapeDtypeStruct((BSZ, NSEG), jnp.float32),
        ),
        scratch_types=[
            pltpu.VMEM((NIR, IROW), jnp.int32),
            pltpu.VMEM((HALF,), jnp.float32),
            pltpu.VMEM((CHUNK, HID), jnp.float32),
            pltpu.VMEM((CHUNK, HID), jnp.float32),
            pltpu.VMEM((NSEG * 16,), jnp.float32),
            pltpu.VMEM((NSEG * 16,), jnp.float32),
            pltpu.VMEM((NSEG * 16,), jnp.float32),
            pltpu.VMEM((NSEG * 8,), jnp.float32),
            pltpu.VMEM((NSEG,), jnp.float32),
            pltpu.VMEM_SHARED((8 * NSEG, HID), jnp.float32),
            pltpu.VMEM_SHARED((16, NSEG * 8), jnp.float32),
            pltpu.VMEM_SHARED((16, NSEG * 8), jnp.float32),
            pltpu.SemaphoreType.DMA,
            pltpu.SemaphoreType.DMA,
            pltpu.SemaphoreType.DMA,
            pltpu.SemaphoreType.DMA,
        ],
    )(_tagop_body)
    idx2 = index.reshape(BSZ, SEQ // IROW, IROW)
    return fn(values, scores, idx2)


def kernel(values, scores, index):
    return _tagop(values, scores, index)
